# baseline (speedup 1.0000x reference)
import functools

import jax
import jax.numpy as jnp
from jax import lax
from jax.experimental import pallas as pl
from jax.experimental.pallas import tpu as pltpu
from jax.experimental.pallas import tpu_sc as plsc

D_MODEL = 2048
N_EXPERTS = 64
TOP_K = 8
BLOCK_TOKENS = 2048

_SC_INFO = plsc.get_sparse_core_info()
_NC, _NS, _L = _SC_INFO.num_cores, _SC_INFO.num_subcores, _SC_INFO.num_lanes
_NW = _NC * _NS
_CHUNK = 64


def _probs_block(x_ref, w_ref, probs_ref):
    x = x_ref[...]
    w = w_ref[...]
    logits = jnp.dot(x, w, preferred_element_type=jnp.float32)
    m = jnp.max(logits, axis=-1, keepdims=True)
    e = jnp.exp(logits - m)
    probs_ref[...] = e / jnp.sum(e, axis=-1, keepdims=True)


def _tc_probs(x, W_t):
    n_tokens = x.shape[0]
    return pl.pallas_call(
        _probs_block,
        grid=(n_tokens // BLOCK_TOKENS,),
        in_specs=[
            pl.BlockSpec((BLOCK_TOKENS, D_MODEL), lambda i: (i, 0)),
            pl.BlockSpec((D_MODEL, N_EXPERTS), lambda i: (0, 0)),
        ],
        out_specs=pl.BlockSpec((BLOCK_TOKENS, N_EXPERTS), lambda i: (i, 0)),
        out_shape=jax.ShapeDtypeStruct((n_tokens, N_EXPERTS), jnp.float32),
    )(x, W_t)


def _merge16(ak, ai, bk, bi):
    rbk = lax.rev(bk, (0,))
    rbi = lax.rev(bi, (0,))
    take = ak >= rbk
    ck = jnp.where(take, ak, rbk)
    ci = jnp.where(take, ai, rbi)
    return plsc.sort_key_val(ck, ci, descending=True)


def _sc_body(probs_hbm, out_w_hbm, out_i_hbm, pv, ow, oi, sem):
    n_tokens = probs_hbm.shape[0]
    per_w = n_tokens // _NW
    wid = lax.axis_index("s") * _NC + lax.axis_index("c")
    tok_base = wid * per_w

    lane = lax.iota(jnp.int32, _L)
    mask8 = lane < TOP_K

    for c in range(per_w // _CHUNK):
        c0 = tok_base + c * _CHUNK
        pltpu.sync_copy(probs_hbm.at[pl.ds(c0, _CHUNK), :], pv)

        def token(t, _):
            chunks = []
            for k in range(N_EXPERTS // _L):
                keys = pv[t, pl.ds(k * _L, _L)]
                idx = lane + (k * _L)
                chunks.append(plsc.sort_key_val(keys, idx, descending=True))
            m01 = _merge16(*chunks[0], *chunks[1])
            m23 = _merge16(*chunks[2], *chunks[3])
            fk, fi = _merge16(*m01, *m23)
            s8 = jnp.sum(jnp.where(mask8, fk, 0.0))
            wv = fk / (s8 + 1e-9)
            off = t * TOP_K
            plsc.store_compressed(ow.at[pl.ds(off, _L)], wv, mask=mask8)
            plsc.store_compressed(oi.at[pl.ds(off, _L)], fi, mask=mask8)
            return _

        lax.fori_loop(0, _CHUNK, token, 0)
        out0 = c0 * TOP_K
        n_out = _CHUNK * TOP_K
        pltpu.sync_copy(ow.at[pl.ds(0, n_out)], out_w_hbm.at[pl.ds(out0, n_out)])
        pltpu.sync_copy(oi.at[pl.ds(0, n_out)], out_i_hbm.at[pl.ds(out0, n_out)])


def _sc_topk(probs):
    n_tokens = probs.shape[0]
    flat = n_tokens * TOP_K
    mesh = plsc.VectorSubcoreMesh(core_axis_name="c", subcore_axis_name="s")
    fn = functools.partial(
        pl.kernel,
        out_type=[
            jax.ShapeDtypeStruct((flat,), jnp.float32),
            jax.ShapeDtypeStruct((flat,), jnp.int32),
        ],
        mesh=mesh,
        compiler_params=pltpu.CompilerParams(needs_layout_passes=False),
        scratch_types=[
            pltpu.VMEM((_CHUNK, N_EXPERTS), jnp.float32),
            pltpu.VMEM((_CHUNK * TOP_K + _L,), jnp.float32),
            pltpu.VMEM((_CHUNK * TOP_K + _L,), jnp.int32),
            pltpu.SemaphoreType.DMA,
        ],
    )(_sc_body)
    out_w, out_i = fn(probs)
    return (out_w.reshape(n_tokens, TOP_K), out_i.reshape(n_tokens, TOP_K))


def kernel(x, W_t):
    probs = _tc_probs(x, W_t)
    top_w, top_i = _sc_topk(probs)
    return top_w, top_i.astype(jnp.int64)


# speedup vs baseline: 2.7244x; 2.7244x over previous
import jax
import jax.numpy as jnp
from jax.experimental import pallas as pl
from jax.experimental.pallas import tpu as pltpu

D_MODEL = 2048
N_EXPERTS = 64
TOP_K = 8
BLOCK_TOKENS = 1024
N_BUF = 2


def _router_body(x_hbm, w_ref, out_w_ref, out_i_ref, xbuf, sem):
    n_tokens = x_hbm.shape[0]
    n_blocks = n_tokens // BLOCK_TOKENS
    w = w_ref[...]

    def x_copy(i, slot):
        return pltpu.make_async_copy(
            x_hbm.at[pl.ds(i * BLOCK_TOKENS, BLOCK_TOKENS), :],
            xbuf.at[slot],
            sem.at[slot],
        )

    for j in range(N_BUF - 1):
        x_copy(j, j % N_BUF).start()
    for i in range(n_blocks):
        if i + N_BUF - 1 < n_blocks:
            x_copy(i + N_BUF - 1, (i + N_BUF - 1) % N_BUF).start()
        x_copy(i, i % N_BUF).wait()
        x = xbuf[i % N_BUF]

        logits = jax.lax.dot_general(
            w, x, (((0,), (1,)), ((), ())),
            preferred_element_type=jnp.float32)

        m = jnp.max(logits, axis=0, keepdims=True)
        e = jnp.exp(logits - m)
        s = jnp.sum(e, axis=0, keepdims=True)
        probs = e / s

        lane = jax.lax.broadcasted_iota(
            jnp.int32, probs.shape, 0).astype(jnp.float32)
        vals = []
        idxs = []
        p = probs
        for k in range(TOP_K):
            mk = jnp.max(p, axis=0, keepdims=True)
            ik = jnp.min(jnp.where(p == mk, lane, float(N_EXPERTS)),
                         axis=0, keepdims=True)
            vals.append(mk)
            idxs.append(ik)
            if k + 1 < TOP_K:
                p = jnp.where(lane == ik, -1.0, p)

        top_w = jnp.concatenate(vals, axis=0)
        top_i = jnp.concatenate(idxs, axis=0)
        top_w = top_w / (jnp.sum(top_w, axis=0, keepdims=True) + 1e-9)

        cols = pl.ds(i * BLOCK_TOKENS, BLOCK_TOKENS)
        out_w_ref[:, cols] = top_w
        out_i_ref[:, cols] = top_i.astype(jnp.int32)


def kernel(x, W_t):
    n_tokens = x.shape[0]
    out_w_t, out_i_t = pl.pallas_call(
        _router_body,
        in_specs=[
            pl.BlockSpec(memory_space=pltpu.HBM),
            pl.BlockSpec(memory_space=pltpu.VMEM),
        ],
        out_specs=[
            pl.BlockSpec(memory_space=pltpu.VMEM),
            pl.BlockSpec(memory_space=pltpu.VMEM),
        ],
        out_shape=[
            jax.ShapeDtypeStruct((TOP_K, n_tokens), jnp.float32),
            jax.ShapeDtypeStruct((TOP_K, n_tokens), jnp.int32),
        ],
        scratch_shapes=[
            pltpu.VMEM((N_BUF, BLOCK_TOKENS, D_MODEL), jnp.float32),
            pltpu.SemaphoreType.DMA((N_BUF,)),
        ],
    )(x, W_t)
    return out_w_t.T, out_i_t.T.astype(jnp.int64)
